# baseline (speedup 1.0000x reference)
import functools

import jax
import jax.numpy as jnp
from jax import lax
from jax.experimental import pallas as pl
from jax.experimental.pallas import tpu as pltpu
from jax.experimental.pallas import tpu_sc as plsc

BATCH = 1024
SEQ = 128
VOCAB = 100000
D = 128

NC = 2
NS = 16
NW = NC * NS
TOK = BATCH * SEQ
TPW = TOK // NW
G = 128
KCH = TPW // G
NRING = 4
GA = 2
OB = 2


def _sc_body(ids_hbm, mask_hbm, table_hbm, out_hbm,
             idx_v, mask_iv, srcpos, comp2d, crows, obuf, gs, os):
    c = lax.axis_index("c")
    s = lax.axis_index("s")
    wid = s * NC + c
    base = wid * TPW

    pltpu.sync_copy(ids_hbm.at[pl.ds(wid * KCH, KCH)], idx_v)
    pltpu.sync_copy(mask_hbm.at[pl.ds(wid * KCH, KCH)], mask_iv)

    for j in range(D // 16):
        crows[0, pl.ds(j * 16, 16)] = jnp.zeros((16,), jnp.float32)

    def p1(i, cnt):
        r = i >> 3
        slc = pl.ds((i & 7) * 16, 16)
        ids16 = idx_v[r, slc]
        m16 = mask_iv[r, slc]
        inc = plsc.cumsum(m16)
        pos16 = (cnt + inc) - m16
        plsc.store_scatter(
            comp2d,
            [lax.shift_right_logical(pos16, 7), lax.bitwise_and(pos16, G - 1)],
            ids16, mask=m16 != 0)
        srcpos[pl.ds(i * 16, 16)] = pos16
        return cnt + jnp.sum(m16)

    n = lax.fori_loop(0, TPW // 16, p1, jnp.int32(0))
    srcpos[pl.ds(TPW, 16)] = jnp.full((16,), 0, jnp.int32) + n

    dup = jnp.full((16,), 0, jnp.int32) + comp2d[0, pl.ds(0, 16)][0]
    for k in range(8):
        p = (n + 16 * k) + lax.iota(jnp.int32, 16)
        plsc.store_scatter(
            comp2d,
            [lax.shift_right_logical(p, 7), lax.bitwise_and(p, G - 1)],
            dup)
    n_g = lax.shift_right_logical(n + G - 1, 7)

    for k in range(GA + 1):
        @pl.when(k < n_g)
        def _():
            pltpu.async_copy(table_hbm.at[comp2d.at[k]],
                             crows.at[pl.ds(k * G, G)], gs[k])
    nm1 = jnp.maximum(n - 1, 0)

    def half(i, carry):
        qiss, qdone = carry
        for h in range(OB):
            g = i * OB + h
            snext = srcpos[pl.ds((g + 1) * G, 16)][0]
            qneed = lax.shift_right_logical(snext + G - 1, 7) - 1

            tgt = jnp.minimum(qneed + GA, n_g - 1)
            qq = qiss + 1

            @pl.when(qiss < tgt)
            def _():
                for b in range(NRING):
                    @pl.when((qq & (NRING - 1)) == b)
                    def _():
                        pltpu.async_copy(table_hbm.at[comp2d.at[qq]],
                                         crows.at[pl.ds(b * G, G)], gs[b])
            qiss = jnp.where(qiss < tgt, qq, qiss)

            wq = qdone + 1

            @pl.when(qdone < qneed)
            def _():
                for b in range(NRING):
                    @pl.when((wq & (NRING - 1)) == b)
                    def _():
                        pltpu.make_async_copy(table_hbm.at[comp2d.at[wq]],
                                              crows.at[pl.ds(b * G, G)],
                                              gs[b]).wait()
            qdone = jnp.where(qdone < qneed, wq, qdone)

            obase = h * G

            @pl.when(g >= OB)
            def _():
                pltpu.make_async_copy(
                    obuf.at[pl.ds(obase, G)],
                    out_hbm.at[pl.ds(base, G)], os[h]).wait()

            def chunk(cc, carry2):
                off = g * G + cc * 16
                sv = srcpos[pl.ds(off, 16)]
                svn = srcpos[pl.ds(off + 1, 16)]
                for tt in range(16):
                    src = sv[tt]
                    mv = jnp.full((16,), jnp.float32(0)) + (
                        svn[tt] - src).astype(jnp.float32)
                    srow = jnp.minimum(src, nm1) & (NRING * G - 1)
                    for j in range(D // 16):
                        slj = pl.ds(j * 16, 16)
                        obuf[obase + cc * 16 + tt, slj] = (
                            crows[srow, slj] * mv)
                return carry2

            lax.fori_loop(0, G // 16, chunk, 0)

            pltpu.async_copy(
                obuf.at[pl.ds(obase, G)],
                out_hbm.at[pl.ds(base + g * G, G)], os[h])
        return (qiss, qdone)

    lax.fori_loop(0, KCH // OB, half, (jnp.minimum(jnp.int32(GA), n_g - 1),
                                       jnp.int32(-1)))

    for h in range(OB):
        pltpu.make_async_copy(
            obuf.at[pl.ds(h * G, G)],
            out_hbm.at[pl.ds(base, G)], os[h]).wait()


def _sc_entry(ids_hbm, mask_hbm, table_hbm, out_hbm, *scratch):
    gs = scratch[6:6 + NRING]
    os = scratch[6 + NRING:6 + NRING + OB]
    _sc_body(ids_hbm, mask_hbm, table_hbm, out_hbm,
             scratch[0], scratch[1], scratch[2], scratch[3],
             scratch[4], scratch[5], gs, os)


@jax.jit
def _sc_call(ids, mask, table):
    mesh = plsc.VectorSubcoreMesh(core_axis_name="c", subcore_axis_name="s")
    kfn = functools.partial(
        pl.kernel,
        mesh=mesh,
        out_type=jax.ShapeDtypeStruct((TOK, D), jnp.float32),
        scratch_types=[
            pltpu.VMEM((KCH, G), jnp.int32),
            pltpu.VMEM((KCH, G), jnp.int32),
            pltpu.VMEM((TPW + 16,), jnp.int32),
            pltpu.VMEM((KCH + 1, G), jnp.int32),
            pltpu.VMEM((NRING * G, D), jnp.float32),
            pltpu.VMEM((OB * G, D), jnp.float32),
        ] + [pltpu.SemaphoreType.DMA for _ in range(NRING + OB)],
        compiler_params=pltpu.CompilerParams(needs_layout_passes=False),
    )(_sc_entry)
    return kfn(ids, mask, table)


def kernel(input_ids, attention_mask, embedding_table):
    out = _sc_call(input_ids, attention_mask, embedding_table)
    return out.reshape(BATCH, SEQ, D)


# speedup vs baseline: 1.0106x; 1.0106x over previous
import functools

import jax
import jax.numpy as jnp
from jax import lax
from jax.experimental import pallas as pl
from jax.experimental.pallas import tpu as pltpu
from jax.experimental.pallas import tpu_sc as plsc

BATCH = 1024
SEQ = 128
VOCAB = 100000
D = 128

NC = 2
NS = 16
NW = NC * NS
TOK = BATCH * SEQ
TPW = TOK // NW
G = 128
KCH = TPW // G
NRING = 4
GA = 2
OB = 2


def _sc_body(ids_hbm, mask_hbm, table_hbm, out_hbm,
             idx_v, mask_iv, srcpos, comp2d, crows, obuf, gs, os):
    c = lax.axis_index("c")
    s = lax.axis_index("s")
    wid = s * NC + c
    base = wid * TPW

    pltpu.sync_copy(ids_hbm.at[pl.ds(wid * KCH, KCH)], idx_v)
    pltpu.sync_copy(mask_hbm.at[pl.ds(wid * KCH, KCH)], mask_iv)

    for j in range(D // 16):
        crows[0, pl.ds(j * 16, 16)] = jnp.zeros((16,), jnp.float32)

    def p1(i, cnt):
        r = i >> 3
        slc = pl.ds((i & 7) * 16, 16)
        ids16 = idx_v[r, slc]
        m16 = mask_iv[r, slc]
        inc = plsc.cumsum(m16)
        pos16 = (cnt + inc) - m16
        plsc.store_scatter(
            comp2d,
            [lax.shift_right_logical(pos16, 7), lax.bitwise_and(pos16, G - 1)],
            ids16, mask=m16 != 0)
        srcpos[pl.ds(i * 16, 16)] = pos16
        return cnt + jnp.sum(m16)

    n = lax.fori_loop(0, TPW // 16, p1, jnp.int32(0))
    srcpos[pl.ds(TPW, 16)] = jnp.full((16,), 0, jnp.int32) + n

    dup = jnp.full((16,), 0, jnp.int32) + comp2d[0, pl.ds(0, 16)][0]
    for k in range(8):
        p = (n + 16 * k) + lax.iota(jnp.int32, 16)
        plsc.store_scatter(
            comp2d,
            [lax.shift_right_logical(p, 7), lax.bitwise_and(p, G - 1)],
            dup)
    n_g = lax.shift_right_logical(n + G - 1, 7)

    for k in range(GA + 1):
        @pl.when(k < n_g)
        def _():
            pltpu.async_copy(table_hbm.at[comp2d.at[k]],
                             crows.at[pl.ds(k * G, G)], gs[k])
    nm1 = jnp.maximum(n - 1, 0)

    def half(i, carry):
        qiss, qdone = carry
        for h in range(OB):
            g = i * OB + h
            snext = srcpos[pl.ds((g + 1) * G, 16)][0]
            qneed = lax.shift_right_logical(snext + G - 1, 7) - 1

            tgt = jnp.minimum(qneed + GA, n_g - 1)
            qq = qiss + 1

            @pl.when(qiss < tgt)
            def _():
                for b in range(NRING):
                    @pl.when((qq & (NRING - 1)) == b)
                    def _():
                        pltpu.async_copy(table_hbm.at[comp2d.at[qq]],
                                         crows.at[pl.ds(b * G, G)], gs[b])
            qiss = jnp.where(qiss < tgt, qq, qiss)

            wq = qdone + 1

            @pl.when(qdone < qneed)
            def _():
                for b in range(NRING):
                    @pl.when((wq & (NRING - 1)) == b)
                    def _():
                        pltpu.make_async_copy(table_hbm.at[comp2d.at[wq]],
                                              crows.at[pl.ds(b * G, G)],
                                              gs[b]).wait()
            qdone = jnp.where(qdone < qneed, wq, qdone)

            obase = h * G

            @pl.when(g >= OB)
            def _():
                pltpu.make_async_copy(
                    obuf.at[pl.ds(obase, G)],
                    out_hbm.at[pl.ds(base, G)], os[h]).wait()

            def chunk(cc, carry2):
                off = g * G + cc * 16
                sv = srcpos[pl.ds(off, 16)]
                svn = srcpos[pl.ds(off + 1, 16)]
                m16f = (svn - sv).astype(jnp.float32)
                srow16 = jnp.minimum(sv, nm1) & (NRING * G - 1)
                for tt in range(16):
                    lane = jnp.full((16,), tt, jnp.int32)
                    rvec = srow16.at[lane].get(mode="promise_in_bounds")
                    mv = m16f.at[lane].get(mode="promise_in_bounds")
                    for j in range(D // 16):
                        colv = lax.iota(jnp.int32, 16) + (j * 16)
                        row = plsc.load_gather(crows, [rvec, colv])
                        obuf[obase + cc * 16 + tt, pl.ds(j * 16, 16)] = (
                            row * mv)
                return carry2

            lax.fori_loop(0, G // 16, chunk, 0)

            pltpu.async_copy(
                obuf.at[pl.ds(obase, G)],
                out_hbm.at[pl.ds(base + g * G, G)], os[h])
        return (qiss, qdone)

    lax.fori_loop(0, KCH // OB, half, (jnp.minimum(jnp.int32(GA), n_g - 1),
                                       jnp.int32(-1)))

    for h in range(OB):
        pltpu.make_async_copy(
            obuf.at[pl.ds(h * G, G)],
            out_hbm.at[pl.ds(base, G)], os[h]).wait()


def _sc_entry(ids_hbm, mask_hbm, table_hbm, out_hbm, *scratch):
    gs = scratch[6:6 + NRING]
    os = scratch[6 + NRING:6 + NRING + OB]
    _sc_body(ids_hbm, mask_hbm, table_hbm, out_hbm,
             scratch[0], scratch[1], scratch[2], scratch[3],
             scratch[4], scratch[5], gs, os)


@jax.jit
def _sc_call(ids, mask, table):
    mesh = plsc.VectorSubcoreMesh(core_axis_name="c", subcore_axis_name="s")
    kfn = functools.partial(
        pl.kernel,
        mesh=mesh,
        out_type=jax.ShapeDtypeStruct((TOK, D), jnp.float32),
        scratch_types=[
            pltpu.VMEM((KCH, G), jnp.int32),
            pltpu.VMEM((KCH, G), jnp.int32),
            pltpu.VMEM((TPW + 16,), jnp.int32),
            pltpu.VMEM((KCH + 1, G), jnp.int32),
            pltpu.VMEM((NRING * G, D), jnp.float32),
            pltpu.VMEM((OB * G, D), jnp.float32),
        ] + [pltpu.SemaphoreType.DMA for _ in range(NRING + OB)],
        compiler_params=pltpu.CompilerParams(needs_layout_passes=False),
    )(_sc_entry)
    return kfn(ids, mask, table)


def kernel(input_ids, attention_mask, embedding_table):
    out = _sc_call(input_ids, attention_mask, embedding_table)
    return out.reshape(BATCH, SEQ, D)


# speedup vs baseline: 2.5545x; 2.5277x over previous
import functools

import jax
import jax.numpy as jnp
from jax import lax
from jax.experimental import pallas as pl
from jax.experimental.pallas import tpu as pltpu
from jax.experimental.pallas import tpu_sc as plsc

BATCH = 1024
SEQ = 128
VOCAB = 100000
D = 128

NC = 2
NS = 16
NW = NC * NS
TOK = BATCH * SEQ
TPW = TOK // NW
G = 128
KCH = TPW // G
NBUF = 7
GA = 3


def _sc_body(ids_hbm, mask_hbm, table_hbm, out_hbm, idx_v, mask_iv, mask_v,
             rows, gs, os):
    c = lax.axis_index("c")
    s = lax.axis_index("s")
    wid = s * NC + c
    base = wid * TPW

    pltpu.sync_copy(ids_hbm.at[pl.ds(wid * KCH, KCH)], idx_v)
    pltpu.sync_copy(mask_hbm.at[pl.ds(wid * KCH, KCH)], mask_iv)

    def cvt_row(r, carry):
        for cgrp in range(G // 16):
            sl = pl.ds(cgrp * 16, 16)
            mask_v[pl.ds(r * G + cgrp * 16, 16)] = mask_iv[r, sl].astype(
                jnp.float32)
        return carry

    lax.fori_loop(0, KCH, cvt_row, 0)

    ghandle = [None] * NBUF
    ohandle = [None] * NBUF

    def multiply(rv, g):
        def tok(t, carry2):
            midx = jnp.full((16,), 0, jnp.int32) + (g * G + t)
            m = plsc.load_gather(mask_v, [midx])
            for j in range(D // 16):
                sl = pl.ds(j * 16, 16)
                rv[t, sl] = rv[t, sl] * m
            return carry2

        lax.fori_loop(0, G, tok, 0)

    def start_gather(g):
        b = g % NBUF
        ghandle[b] = pltpu.async_copy(
            table_hbm.at[idx_v.at[g]], rows[b], gs[b])

    for g in range(GA + 1):
        start_gather(g)

    for g in range(KCH):
        b = g % NBUF
        gn = g + GA + 1
        if gn < KCH:
            bn = gn % NBUF
            if ohandle[bn] is not None:
                ohandle[bn].wait()
                ohandle[bn] = None
            start_gather(gn)
        ghandle[b].wait()
        multiply(rows[b], g)
        ohandle[b] = pltpu.async_copy(
            rows[b], out_hbm.at[pl.ds(base + g * G, G)], os[b])
    for b in range(NBUF):
        if ohandle[b] is not None:
            ohandle[b].wait()


def _sc_entry(ids_hbm, mask_hbm, table_hbm, out_hbm, *scratch):
    rows = scratch[3:3 + NBUF]
    gs = scratch[3 + NBUF:3 + 2 * NBUF]
    os = scratch[3 + 2 * NBUF:3 + 3 * NBUF]
    _sc_body(ids_hbm, mask_hbm, table_hbm, out_hbm, scratch[0], scratch[1],
             scratch[2], rows, gs, os)


@jax.jit
def _sc_call(ids, mask, table):
    mesh = plsc.VectorSubcoreMesh(core_axis_name="c", subcore_axis_name="s")
    kfn = functools.partial(
        pl.kernel,
        mesh=mesh,
        out_type=jax.ShapeDtypeStruct((TOK, D), jnp.float32),
        scratch_types=[
            pltpu.VMEM((KCH, G), jnp.int32),
            pltpu.VMEM((KCH, G), jnp.int32),
            pltpu.VMEM((TPW,), jnp.float32),
        ] + [pltpu.VMEM((G, D), jnp.float32) for _ in range(NBUF)]
          + [pltpu.SemaphoreType.DMA for _ in range(2 * NBUF)],
        compiler_params=pltpu.CompilerParams(needs_layout_passes=False),
    )(_sc_entry)
    return kfn(ids, mask, table)


def kernel(input_ids, attention_mask, embedding_table):
    out = _sc_call(input_ids, attention_mask, embedding_table)
    return out.reshape(BATCH, SEQ, D)
